# baseline (speedup 1.0000x reference)
import functools

import jax
import jax.numpy as jnp
from jax import lax
from jax.experimental import pallas as pl
from jax.experimental.pallas import tpu as pltpu
from jax.experimental.pallas import tpu_sc as plsc

N = 10000
E = 320000
D = 128
N_LAYER = 4
N_INNER = 2
BN_EPS = 1e-5

PAD_ROWS = 112
N_PAD = N + PAD_ROWS

CHUNK = 128
GCH = 32


def _sc_counts():
    try:
        info = plsc.get_sparse_core_info()
        return info.num_cores, info.num_subcores
    except Exception:
        return 2, 16


NBUF = 6
LANES = 16
assert CHUNK % GCH == 0


def _make_sc_aggregate(nc, ns):
    nw = nc * ns
    nchunk = -(-E // (nw * CHUNK * 8)) * 8
    edges_per_w = nchunk * CHUNK
    e_pad = nw * edges_per_w
    rows_per_tile = N_PAD // ns
    ngroup = edges_per_w // (GCH * NBUF)

    mesh = plsc.VectorSubcoreMesh(core_axis_name="c", subcore_axis_name="s")

    @functools.partial(
        pl.kernel,
        out_type=jax.ShapeDtypeStruct((nc * N_PAD, D), jnp.float32),
        mesh=mesh,
        scratch_types=(
            [pltpu.VMEM((nchunk, CHUNK), jnp.int32)] +
            [pltpu.VMEM((GCH, D), jnp.float32)] * NBUF +
            [pltpu.VMEM((GCH,), jnp.int32)] * (2 * NBUF) +
            [pltpu.VMEM_SHARED((N_PAD, D), jnp.float32)] +
            [pltpu.SemaphoreType.DMA] * (2 * NBUF)
        ),
    )
    def agg_kernel(h_hbm, se_hbm, zeros_hbm, out_hbm, pbig, *scr):
        rows = scr[:NBUF]
        sidx = scr[NBUF:2 * NBUF]
        didx = scr[2 * NBUF:3 * NBUF]
        agg_sh = scr[3 * NBUF]
        sem_g = scr[3 * NBUF + 1:3 * NBUF + 1 + NBUF]
        sem_s = scr[3 * NBUF + 1 + NBUF:]
        cid = lax.axis_index("c")
        sid = lax.axis_index("s")
        wid = sid * nc + cid

        r0 = sid * rows_per_tile
        pltpu.async_copy(se_hbm.at[pl.ds(wid * nchunk, nchunk)], pbig,
                         sem_g[0])
        pltpu.async_copy(zeros_hbm.at[pl.ds(r0, rows_per_tile)],
                         agg_sh.at[pl.ds(r0, rows_per_tile)], sem_s[0])
        pltpu.make_async_copy(se_hbm.at[pl.ds(wid * nchunk, nchunk)], pbig,
                              sem_g[0]).wait()
        pltpu.make_async_copy(zeros_hbm.at[pl.ds(r0, rows_per_tile)],
                              agg_sh.at[pl.ds(r0, rows_per_tile)],
                              sem_s[0]).wait()
        plsc.subcore_barrier()

        def group(g, carry):
            for bb in range(NBUF):
                row = g * (NBUF * GCH // CHUNK) + (bb * GCH) // CHUNK
                off = (bb * GCH) % CHUNK

                @pl.when(g > 0)
                def _():
                    pltpu.make_async_copy(
                        rows[bb], agg_sh.at[didx[bb]], sem_s[bb]).wait()

                for k in range(GCH // LANES):
                    p = pbig[row, pl.ds(off + k * LANES, LANES)]
                    sl = pl.ds(k * LANES, LANES)
                    sidx[bb][sl] = lax.bitwise_and(p, 0xFFFF)
                    didx[bb][sl] = lax.shift_right_logical(p, 16)
                pltpu.async_copy(h_hbm.at[sidx[bb]], rows[bb], sem_g[bb])
            for bb in range(NBUF):
                pltpu.make_async_copy(
                    h_hbm.at[sidx[bb]], rows[bb], sem_g[bb]).wait()
                pltpu.async_copy(rows[bb], agg_sh.at[didx[bb]],
                                 sem_s[bb], add=True)
            return carry

        lax.fori_loop(0, ngroup, group, 0)
        for bb in range(NBUF):
            pltpu.make_async_copy(
                rows[bb], agg_sh.at[didx[bb]], sem_s[bb]).wait()
        plsc.subcore_barrier()

        pltpu.sync_copy(agg_sh.at[pl.ds(r0, rows_per_tile)],
                        out_hbm.at[pl.ds(cid * N_PAD + r0, rows_per_tile)])

    return agg_kernel, e_pad


def _dense_body(eps_ref, h_ref, parts_ref, w_ref, gl_ref, bl_ref,
                go_ref, bo_ref, out_ref, *, nc, residual):
    h = h_ref[:N]
    agg = parts_ref[:N]
    for c in range(1, nc):
        agg = agg + parts_ref[c * N_PAD:c * N_PAD + N]
    def bn_relu(hh, gamma, beta):
        s1 = jnp.sum(hh, axis=0, keepdims=True)
        s2 = jnp.sum(hh * hh, axis=0, keepdims=True)
        mu = s1 * (1.0 / N)
        var = s2 * (1.0 / N) - mu * mu
        a = gamma * lax.rsqrt(var + BN_EPS)
        c = beta - a * mu
        return jnp.maximum(a * hh + c, 0.0)

    hh = (1.0 + eps_ref[0]) * h + agg
    for j in range(N_INNER):
        hh = lax.dot_general(hh, w_ref[j], (((1,), (1,)), ((), ())),
                             preferred_element_type=jnp.float32)
        hh = bn_relu(hh, gl_ref[j], bl_ref[j])
    hh = bn_relu(hh, go_ref[:], bo_ref[:])
    if residual:
        hh = hh + h
    out_ref[:N] = hh
    out_ref[N:] = jnp.zeros((N_PAD - N, D), jnp.float32)


def _make_dense_call(nc, residual):
    return pl.pallas_call(
        functools.partial(_dense_body, nc=nc, residual=residual),
        out_shape=jax.ShapeDtypeStruct((N_PAD, D), jnp.float32),
        in_specs=[pl.BlockSpec(memory_space=pltpu.SMEM)] +
                 [pl.BlockSpec(memory_space=pltpu.VMEM)] * 7,
        out_specs=pl.BlockSpec(memory_space=pltpu.VMEM),
    )


def kernel(x, edge_index, eps, W, b, gamma_l, beta_l, gamma_o, beta_o):
    nc, ns = _sc_counts()
    agg_call, e_pad = _make_sc_aggregate(nc, ns)

    src = edge_index[0].astype(jnp.int32)
    dst = edge_index[1].astype(jnp.int32)
    npad_e = e_pad - E
    pad_idx = N + (jnp.arange(npad_e, dtype=jnp.int32) % PAD_ROWS)
    src_p = jnp.concatenate([src, pad_idx])
    dst_p = jnp.concatenate([dst, pad_idx])
    se_p = (src_p | (dst_p << 16)).reshape(-1, CHUNK)

    zeros_full = jnp.zeros((N_PAD, D), jnp.float32)
    h = jnp.concatenate([x.astype(jnp.float32),
                         jnp.zeros((PAD_ROWS, D), jnp.float32)])

    del b
    gl3 = gamma_l.reshape(N_LAYER, N_INNER, 1, D)
    bl3 = beta_l.reshape(N_LAYER, N_INNER, 1, D)
    go2 = gamma_o.reshape(N_LAYER, 1, D)
    bo2 = beta_o.reshape(N_LAYER, 1, D)

    dense0 = _make_dense_call(nc, residual=False)
    dense = _make_dense_call(nc, residual=True)
    for l in range(N_LAYER):
        parts = agg_call(h, se_p, zeros_full)
        dc = dense0 if l == 0 else dense
        h = dc(eps[l].reshape(1), h, parts,
               W[l], gl3[l], bl3[l], go2[l], bo2[l])
    return h[:N]


# speedup vs baseline: 1.0258x; 1.0258x over previous
import functools

import jax
import jax.numpy as jnp
from jax import lax
from jax.experimental import pallas as pl
from jax.experimental.pallas import tpu as pltpu
from jax.experimental.pallas import tpu_sc as plsc

N = 10000
E = 320000
D = 128
N_LAYER = 4
N_INNER = 2
BN_EPS = 1e-5

PAD_ROWS = 112
N_PAD = N + PAD_ROWS

CHUNK = 128
GCH = 32


def _sc_counts():
    try:
        info = plsc.get_sparse_core_info()
        return info.num_cores, info.num_subcores
    except Exception:
        return 2, 16


NBUF = 8
LANES = 16
assert CHUNK % GCH == 0


def _make_sc_aggregate(nc, ns):
    nw = nc * ns
    nchunk = -(-E // (nw * CHUNK * 8)) * 8
    edges_per_w = nchunk * CHUNK
    e_pad = nw * edges_per_w
    rows_per_tile = N_PAD // ns
    ngroup = edges_per_w // (GCH * NBUF)

    mesh = plsc.VectorSubcoreMesh(core_axis_name="c", subcore_axis_name="s")

    @functools.partial(
        pl.kernel,
        out_type=jax.ShapeDtypeStruct((nc * N_PAD, D), jnp.float32),
        mesh=mesh,
        scratch_types=(
            [pltpu.VMEM((nchunk, CHUNK), jnp.int32)] +
            [pltpu.VMEM((GCH, D), jnp.float32)] * NBUF +
            [pltpu.VMEM((GCH,), jnp.int32)] * (2 * NBUF) +
            [pltpu.VMEM_SHARED((N_PAD, D), jnp.float32)] +
            [pltpu.SemaphoreType.DMA] * (2 * NBUF)
        ),
    )
    def agg_kernel(h_hbm, se_hbm, zeros_hbm, out_hbm, pbig, *scr):
        rows = scr[:NBUF]
        sidx = scr[NBUF:2 * NBUF]
        didx = scr[2 * NBUF:3 * NBUF]
        agg_sh = scr[3 * NBUF]
        sem_g = scr[3 * NBUF + 1:3 * NBUF + 1 + NBUF]
        sem_s = scr[3 * NBUF + 1 + NBUF:]
        cid = lax.axis_index("c")
        sid = lax.axis_index("s")
        wid = sid * nc + cid

        r0 = sid * rows_per_tile
        pltpu.async_copy(se_hbm.at[pl.ds(wid * nchunk, nchunk)], pbig,
                         sem_g[0])
        pltpu.async_copy(zeros_hbm.at[pl.ds(r0, rows_per_tile)],
                         agg_sh.at[pl.ds(r0, rows_per_tile)], sem_s[0])
        pltpu.make_async_copy(se_hbm.at[pl.ds(wid * nchunk, nchunk)], pbig,
                              sem_g[0]).wait()
        pltpu.make_async_copy(zeros_hbm.at[pl.ds(r0, rows_per_tile)],
                              agg_sh.at[pl.ds(r0, rows_per_tile)],
                              sem_s[0]).wait()
        plsc.subcore_barrier()

        def group(g, carry):
            for bb in range(NBUF):
                row = g * (NBUF * GCH // CHUNK) + (bb * GCH) // CHUNK
                off = (bb * GCH) % CHUNK

                @pl.when(g > 0)
                def _():
                    pltpu.make_async_copy(
                        rows[bb], agg_sh.at[didx[bb]], sem_s[bb]).wait()

                for k in range(GCH // LANES):
                    p = pbig[row, pl.ds(off + k * LANES, LANES)]
                    sl = pl.ds(k * LANES, LANES)
                    sidx[bb][sl] = lax.bitwise_and(p, 0xFFFF)
                    didx[bb][sl] = lax.shift_right_logical(p, 16)
                pltpu.async_copy(h_hbm.at[sidx[bb]], rows[bb], sem_g[bb])
            for bb in range(NBUF):
                pltpu.make_async_copy(
                    h_hbm.at[sidx[bb]], rows[bb], sem_g[bb]).wait()
                pltpu.async_copy(rows[bb], agg_sh.at[didx[bb]],
                                 sem_s[bb], add=True)
            return carry

        lax.fori_loop(0, ngroup, group, 0)
        for bb in range(NBUF):
            pltpu.make_async_copy(
                rows[bb], agg_sh.at[didx[bb]], sem_s[bb]).wait()
        plsc.subcore_barrier()

        pltpu.sync_copy(agg_sh.at[pl.ds(r0, rows_per_tile)],
                        out_hbm.at[pl.ds(cid * N_PAD + r0, rows_per_tile)])

    return agg_kernel, e_pad


def _dense_body(eps_ref, h_ref, parts_ref, w_ref, gl_ref, bl_ref,
                go_ref, bo_ref, out_ref, *, nc, residual):
    h = h_ref[:N]
    agg = parts_ref[:N]
    for c in range(1, nc):
        agg = agg + parts_ref[c * N_PAD:c * N_PAD + N]
    def bn_relu(hh, gamma, beta):
        s1 = jnp.sum(hh, axis=0, keepdims=True)
        s2 = jnp.sum(hh * hh, axis=0, keepdims=True)
        mu = s1 * (1.0 / N)
        var = s2 * (1.0 / N) - mu * mu
        a = gamma * lax.rsqrt(var + BN_EPS)
        c = beta - a * mu
        return jnp.maximum(a * hh + c, 0.0)

    hh = (1.0 + eps_ref[0]) * h + agg
    for j in range(N_INNER):
        hh = lax.dot_general(hh, w_ref[j], (((1,), (1,)), ((), ())),
                             preferred_element_type=jnp.float32)
        hh = bn_relu(hh, gl_ref[j], bl_ref[j])
    hh = bn_relu(hh, go_ref[:], bo_ref[:])
    if residual:
        hh = hh + h
    out_ref[:N] = hh
    out_ref[N:] = jnp.zeros((N_PAD - N, D), jnp.float32)


def _make_dense_call(nc, residual):
    return pl.pallas_call(
        functools.partial(_dense_body, nc=nc, residual=residual),
        out_shape=jax.ShapeDtypeStruct((N_PAD, D), jnp.float32),
        in_specs=[pl.BlockSpec(memory_space=pltpu.SMEM)] +
                 [pl.BlockSpec(memory_space=pltpu.VMEM)] * 7,
        out_specs=pl.BlockSpec(memory_space=pltpu.VMEM),
    )


def kernel(x, edge_index, eps, W, b, gamma_l, beta_l, gamma_o, beta_o):
    nc, ns = _sc_counts()
    agg_call, e_pad = _make_sc_aggregate(nc, ns)

    src = edge_index[0].astype(jnp.int32)
    dst = edge_index[1].astype(jnp.int32)
    npad_e = e_pad - E
    pad_idx = N + (jnp.arange(npad_e, dtype=jnp.int32) % PAD_ROWS)
    src_p = jnp.concatenate([src, pad_idx])
    dst_p = jnp.concatenate([dst, pad_idx])
    se_p = (src_p | (dst_p << 16)).reshape(-1, CHUNK)

    zeros_full = jnp.zeros((N_PAD, D), jnp.float32)
    h = jnp.concatenate([x.astype(jnp.float32),
                         jnp.zeros((PAD_ROWS, D), jnp.float32)])

    del b
    gl3 = gamma_l.reshape(N_LAYER, N_INNER, 1, D)
    bl3 = beta_l.reshape(N_LAYER, N_INNER, 1, D)
    go2 = gamma_o.reshape(N_LAYER, 1, D)
    bo2 = beta_o.reshape(N_LAYER, 1, D)

    dense0 = _make_dense_call(nc, residual=False)
    dense = _make_dense_call(nc, residual=True)
    for l in range(N_LAYER):
        parts = agg_call(h, se_p, zeros_full)
        dc = dense0 if l == 0 else dense
        h = dc(eps[l].reshape(1), h, parts,
               W[l], gl3[l], bl3[l], go2[l], bo2[l])
    return h[:N]
